# baseline (speedup 1.0000x reference)
import functools

import jax
import jax.numpy as jnp
from jax import lax
from jax.experimental import pallas as pl
from jax.experimental.pallas import tpu as pltpu
from jax.experimental.pallas import tpu_sc as plsc

NC = 2
NS = 16
NW = NC * NS
LANES = 16


def _make_sc_pool(B, L, D, dtype):
    rows_w = B // NW
    CA = 104
    CB = L - CA
    nvec = D // LANES
    scale = 1.0 / L

    mesh = plsc.VectorSubcoreMesh(
        core_axis_name="c", subcore_axis_name="s",
        num_cores=NC, num_subcores=NS)

    @functools.partial(
        pl.kernel,
        out_type=jax.ShapeDtypeStruct((B, D), dtype),
        mesh=mesh,
        compiler_params=pltpu.CompilerParams(use_tc_tiling_on_sc=False),
        scratch_types=[
            pltpu.VMEM((rows_w * L,), jnp.int32),
            pltpu.VMEM((2, L, D), dtype),
            pltpu.VMEM((rows_w, D), dtype),
            pltpu.SemaphoreType.DMA,
            pltpu.SemaphoreType.DMA,
        ],
    )
    def sc_pool(table_hbm, idx_hbm, out_hbm, idx_v, buf, pooled_v, sem0, sem1):
        wid = lax.axis_index("s") * NC + lax.axis_index("c")
        ibase = wid * (rows_w * L)
        pltpu.sync_copy(idx_hbm.at[pl.ds(ibase, rows_w * L)], idx_v)
        sems = (sem0, sem1)

        def row_copies(r, b):
            o = r * L
            ca = pltpu.make_async_copy(
                table_hbm.at[idx_v.at[pl.ds(o, CA)]],
                buf.at[b, pl.ds(0, CA)], sems[b])
            cb = pltpu.make_async_copy(
                table_hbm.at[idx_v.at[pl.ds(o + CA, CB)]],
                buf.at[b, pl.ds(CA, CB)], sems[b])
            return ca, cb

        def issue(r, b):
            ca, cb = row_copies(r, b)
            ca.start()
            cb.start()

        def wait_row(r, b):
            ca, cb = row_copies(r, b)
            ca.wait()
            cb.wait()

        def acc_row(r, b):
            def jbody(j, accs):
                return tuple(
                    accs[k] + buf[b, j, pl.ds(k * LANES, LANES)]
                    for k in range(nvec))
            z = jnp.zeros((LANES,), dtype)
            accs = lax.fori_loop(0, L, jbody, (z,) * nvec, unroll=8)
            for k in range(nvec):
                pooled_v[r, pl.ds(k * LANES, LANES)] = accs[k] * scale

        issue(0, 0)
        issue(1, 1)

        def obody(rr, carry):
            for b in range(2):
                r = 2 * rr + b
                wait_row(r, b)

                @pl.when(r + 2 < rows_w)
                def _():
                    issue(r + 2, b)

                acc_row(r, b)
            return carry

        lax.fori_loop(0, rows_w // 2, obody, 0)
        pltpu.sync_copy(pooled_v, out_hbm.at[pl.ds(wid * rows_w, rows_w)])

    return sc_pool


def _dense_body(pooled_ref, w1_ref, b1_ref, w2_ref, b2_ref, out_ref):
    p = pooled_ref[...]
    h = jnp.dot(p, w1_ref[...], preferred_element_type=jnp.float32)
    h = jnp.maximum(h + b1_ref[...], 0.0)
    logits = jnp.dot(h, w2_ref[...], preferred_element_type=jnp.float32)
    logits = logits + b2_ref[...]
    m = jnp.max(logits, axis=-1, keepdims=True)
    e = jnp.exp(logits - m)
    out_ref[...] = e / jnp.sum(e, axis=-1, keepdims=True)


def kernel(inputs, emb_table, W1, b1, W2, b2):
    B, L = inputs.shape
    V, D = emb_table.shape
    H = W1.shape[1]
    C = W2.shape[1]
    CP = 128

    idx_flat = inputs.astype(jnp.int32).reshape(-1)
    pooled = _make_sc_pool(B, L, D, emb_table.dtype)(emb_table, idx_flat)

    w2p = jnp.zeros((H, CP), jnp.float32).at[:, :C].set(W2)
    b2p = jnp.full((1, CP), -1e30, jnp.float32).at[0, :C].set(b2)
    b1r = b1.reshape(1, H)

    out = pl.pallas_call(
        _dense_body,
        out_shape=jax.ShapeDtypeStruct((B, CP), jnp.float32),
    )(pooled, W1, b1r, w2p, b2p)
    return out[:, :C]
